# baseline (speedup 1.0000x reference)
import jax
import jax.numpy as jnp
from jax import lax
from jax.experimental import pallas as pl
from jax.experimental.pallas import tpu as pltpu, tpu_sc as plsc

B, N, C = 4096, 101, 50
NC, NS, L = 2, 16, 16
NW = NC * NS
RPW = B // NW
G = RPW // L
OUT_W = 2 + 2 * N


def _step_body(ttm_ref, tw_ref, dem_ref, ct_ref, uc_ref, vis_ref, comp_ref,
               pend_ref, cn_ref, a_ref, head_ref, visout_ref, compout_ref,
               cn_v, a_v, ct_v, uc_v, itt_v, itw_v, idm_v,
               tt_v, sw_v, d_v, isret_v, pend_v, head_v, vis_v, comp_v, sem):
    wid = lax.axis_index("s") * NC + lax.axis_index("c")
    base = wid * RPW

    pltpu.sync_copy(cn_ref.at[pl.ds(base, RPW)], cn_v)
    pltpu.sync_copy(a_ref.at[pl.ds(base, RPW)], a_v)
    pltpu.sync_copy(ct_ref.at[pl.ds(base, RPW)], ct_v)
    pltpu.sync_copy(uc_ref.at[pl.ds(base, RPW)], uc_v)
    pltpu.sync_copy(pend_ref.at[pl.ds(base * C, RPW * C)], pend_v)
    pltpu.sync_copy(vis_ref.at[pl.ds(base, RPW), :], vis_v)
    pltpu.sync_copy(comp_ref.at[pl.ds(base, RPW), :], comp_v)

    lanes = lax.iota(jnp.int32, L)
    for g in range(G):
        sl = pl.ds(g * L, L)
        cn16 = cn_v[sl]
        a16 = a_v[sl]
        gi = base + g * L + lanes
        itt_v[sl] = gi * (N * N) + cn16 * N + a16
        itw_v[sl] = gi * (2 * N) + 2 * a16
        idm_v[sl] = gi * N + a16

    pltpu.async_copy(ttm_ref.at[itt_v], tt_v, sem).wait()
    pltpu.async_copy(tw_ref.at[itw_v], sw_v, sem).wait()
    pltpu.async_copy(dem_ref.at[idm_v], d_v, sem).wait()

    zero_f = jnp.zeros((L,), jnp.float32)
    one_f = jnp.ones((L,), jnp.float32)
    zero_i = jnp.zeros((L,), jnp.int32)
    one_i = jnp.ones((L,), jnp.int32)
    for g in range(G):
        sl = pl.ds(g * L, L)
        cn16 = cn_v[sl]
        a16 = a_v[sl]
        rows = g * L + lanes
        is_ret = (a16 == 0) & (cn16 != 0)
        sst = jnp.maximum(ct_v[sl] + tt_v[sl], sw_v[sl])
        sst = jnp.where(is_ret, zero_f, sst)
        ld = jnp.where(is_ret, zero_f, uc_v[sl] + d_v[sl])
        plsc.store_scatter(head_v, [rows, zero_i], sst)
        plsc.store_scatter(head_v, [rows, one_i], ld)
        nondep = jnp.where(a16 != 0, one_f, zero_f)
        plsc.store_scatter(vis_v, [rows, a16], nondep)
        isdrop = ((a16 & 1) == 0) & (a16 != 0)
        partner = jnp.maximum(a16 - 1, 0)
        plsc.store_scatter(comp_v, [rows, a16], one_f, mask=isdrop)
        plsc.store_scatter(comp_v, [rows, partner], one_f, mask=isdrop)
        isret_v[sl] = jnp.where(is_ret, one_i, zero_i)

    def group_body(g, carry):
        isret16 = isret_v[pl.ds(g * L, L)]
        rows = g * L + lanes
        is_ret_b = isret16 == 1

        @pl.when(jnp.max(isret16) == 1)
        def _reset():
            def col_body(c, inner):
                p16 = plsc.load_gather(pend_v, [rows * C + c])
                pm1 = jnp.maximum(p16 - 1, 0)
                plsc.store_scatter(vis_v, [rows, pm1], zero_f,
                                   mask=(p16 != 0) & is_ret_b)
                return inner

            lax.fori_loop(0, C, col_body, 0)

        return carry

    lax.fori_loop(0, G, group_body, 0)

    pltpu.sync_copy(head_v, head_ref.at[pl.ds(base, RPW), :])
    pltpu.sync_copy(vis_v, visout_ref.at[pl.ds(base, RPW), :])
    pltpu.sync_copy(comp_v, compout_ref.at[pl.ds(base, RPW), :])


def kernel(travel_time_matrix, time_windows, demand, current_time, used_capacity,
           visited, completed, pending_schedule, current_node, action):
    ttm = travel_time_matrix.reshape(-1)
    tw = time_windows.reshape(-1)
    dem = demand.reshape(-1)
    ct = current_time.reshape(-1)
    uc = used_capacity.reshape(-1)
    vis = visited.astype(jnp.float32)
    comp = completed.astype(jnp.float32)
    pend = pending_schedule.astype(jnp.int32).reshape(-1)
    cn = current_node.reshape(-1).astype(jnp.int32)
    a = action.astype(jnp.int32)

    f = pl.kernel(
        _step_body,
        out_type=[jax.ShapeDtypeStruct((B, 2), jnp.float32),
                  jax.ShapeDtypeStruct((B, N), jnp.float32),
                  jax.ShapeDtypeStruct((B, N), jnp.float32)],
        mesh=plsc.VectorSubcoreMesh(core_axis_name="c", subcore_axis_name="s"),
        compiler_params=pltpu.CompilerParams(use_tc_tiling_on_sc=False, needs_layout_passes=False),
        scratch_types=[
            pltpu.VMEM((RPW,), jnp.int32),
            pltpu.VMEM((RPW,), jnp.int32),
            pltpu.VMEM((RPW,), jnp.float32),
            pltpu.VMEM((RPW,), jnp.float32),
            pltpu.VMEM((RPW,), jnp.int32),
            pltpu.VMEM((RPW,), jnp.int32),
            pltpu.VMEM((RPW,), jnp.int32),
            pltpu.VMEM((RPW,), jnp.float32),
            pltpu.VMEM((RPW,), jnp.float32),
            pltpu.VMEM((RPW,), jnp.float32),
            pltpu.VMEM((RPW,), jnp.int32),
            pltpu.VMEM((RPW * C,), jnp.int32),
            pltpu.VMEM((RPW, 2), jnp.float32),
            pltpu.VMEM((RPW, N), jnp.float32),
            pltpu.VMEM((RPW, N), jnp.float32),
            pltpu.SemaphoreType.DMA,
        ],
    )
    head, vis_out, comp_out = f(ttm, tw, dem, ct, uc, vis, comp, pend, cn, a)
    return jnp.concatenate([head, vis_out, comp_out], axis=-1)


# speedup vs baseline: 8.7858x; 8.7858x over previous
import jax
import jax.numpy as jnp
from jax import lax
from jax.experimental import pallas as pl
from jax.experimental.pallas import tpu as pltpu, tpu_sc as plsc

B, N, C = 4096, 101, 50
NC, NS, L = 2, 16, 16
NW = NC * NS
RPW = B // NW
G = RPW // L


def _step_body(ttm_ref, tw_ref, dem_ref, ct_ref, uc_ref, vis_ref, comp_ref,
               pend_ref, cn_ref, a_ref,
               sst_ref, ld_ref, visout_ref, compout_ref,
               cn_v, a_v, ct_v, uc_v, sst_v, ld_v, isret_v,
               slab_v, tw_v, dem_v, pend_v, vis_v, comp_v, sem):
    wid = lax.axis_index("s") * NC + lax.axis_index("c")
    base = wid * RPW

    pltpu.sync_copy(cn_ref.at[pl.ds(base, RPW)], cn_v)
    pltpu.sync_copy(a_ref.at[pl.ds(base, RPW)], a_v)
    pltpu.sync_copy(ct_ref.at[pl.ds(base, RPW)], ct_v)
    pltpu.sync_copy(uc_ref.at[pl.ds(base, RPW)], uc_v)
    pltpu.sync_copy(tw_ref.at[pl.ds(base, RPW), :], tw_v)
    pltpu.sync_copy(dem_ref.at[pl.ds(base, RPW), :], dem_v)
    pltpu.sync_copy(pend_ref.at[pl.ds(base, RPW), :], pend_v)
    pltpu.sync_copy(vis_ref.at[pl.ds(base, RPW), :], vis_v)
    pltpu.sync_copy(comp_ref.at[pl.ds(base, RPW), :], comp_v)

    lanes = lax.iota(jnp.int32, L)
    zero_f = jnp.zeros((L,), jnp.float32)
    one_f = jnp.ones((L,), jnp.float32)
    zero_i = jnp.zeros((L,), jnp.int32)
    one_i = jnp.ones((L,), jnp.int32)

    for g in range(G):
        rows = g * L + lanes
        cn16 = plsc.load_gather(cn_v, [rows])
        a16 = plsc.load_gather(a_v, [rows])

        descs = []
        for j in range(L):
            bi = base + g * L + j
            cnj = cn16[j]
            cn8 = pl.multiple_of((cnj >> 3) << 3, 8)
            descs.append(pltpu.async_copy(
                ttm_ref.at[bi, pl.ds(cn8, 8), :], slab_v.at[j], sem))
        for d in descs:
            d.wait()
        tt16 = plsc.load_gather(slab_v, [lanes, cn16 & 7, a16])

        sw16 = plsc.load_gather(tw_v, [rows, a16])
        d16 = plsc.load_gather(dem_v, [rows, a16])
        ct16 = plsc.load_gather(ct_v, [rows])
        uc16 = plsc.load_gather(uc_v, [rows])

        is_ret = (a16 == 0) & (cn16 != 0)
        sst = jnp.maximum(ct16 + tt16, sw16)
        sst = jnp.where(is_ret, zero_f, sst)
        ld = jnp.where(is_ret, zero_f, uc16 + d16)
        plsc.store_scatter(sst_v, [rows], sst)
        plsc.store_scatter(ld_v, [rows], ld)

        nondep = jnp.where(a16 != 0, one_f, zero_f)
        plsc.store_scatter(vis_v, [rows, a16], nondep)
        isdrop = ((a16 & 1) == 0) & (a16 != 0)
        partner = jnp.maximum(a16 - 1, 0)
        plsc.store_scatter(comp_v, [rows, a16], one_f, mask=isdrop)
        plsc.store_scatter(comp_v, [rows, partner], one_f, mask=isdrop)
        plsc.store_scatter(isret_v, [rows], jnp.where(is_ret, one_i, zero_i))

    def group_body(g, carry):
        rows = g * L + lanes
        isret16 = plsc.load_gather(isret_v, [rows])
        is_ret_b = isret16 == 1

        @pl.when(jnp.max(isret16) == 1)
        def _reset():
            def col_body(c, inner):
                p16 = plsc.load_gather(pend_v, [rows, jnp.full((L,), c, jnp.int32)])
                pm1 = jnp.maximum(p16 - 1, 0)
                plsc.store_scatter(vis_v, [rows, pm1], zero_f,
                                   mask=(p16 != 0) & is_ret_b)
                return inner

            lax.fori_loop(0, C, col_body, 0)

        return carry

    lax.fori_loop(0, G, group_body, 0)

    pltpu.sync_copy(sst_v, sst_ref.at[pl.ds(base, RPW)])
    pltpu.sync_copy(ld_v, ld_ref.at[pl.ds(base, RPW)])
    pltpu.sync_copy(vis_v, visout_ref.at[pl.ds(base, RPW), :])
    pltpu.sync_copy(comp_v, compout_ref.at[pl.ds(base, RPW), :])


def kernel(travel_time_matrix, time_windows, demand, current_time, used_capacity,
           visited, completed, pending_schedule, current_node, action):
    tw0 = time_windows[:, :, 0]
    ct = current_time.reshape(-1)
    uc = used_capacity.reshape(-1)
    vis = visited.astype(jnp.float32)
    comp = completed.astype(jnp.float32)
    pend = pending_schedule.astype(jnp.int32)
    cn = current_node.reshape(-1).astype(jnp.int32)
    a = action.astype(jnp.int32)

    f = pl.kernel(
        _step_body,
        out_type=[jax.ShapeDtypeStruct((B,), jnp.float32),
                  jax.ShapeDtypeStruct((B,), jnp.float32),
                  jax.ShapeDtypeStruct((B, N), jnp.float32),
                  jax.ShapeDtypeStruct((B, N), jnp.float32)],
        mesh=plsc.VectorSubcoreMesh(core_axis_name="c", subcore_axis_name="s"),
        compiler_params=pltpu.CompilerParams(use_tc_tiling_on_sc=True,
                                             needs_layout_passes=False),
        scratch_types=[
            pltpu.VMEM((RPW,), jnp.int32),
            pltpu.VMEM((RPW,), jnp.int32),
            pltpu.VMEM((RPW,), jnp.float32),
            pltpu.VMEM((RPW,), jnp.float32),
            pltpu.VMEM((RPW,), jnp.float32),
            pltpu.VMEM((RPW,), jnp.float32),
            pltpu.VMEM((RPW,), jnp.int32),
            pltpu.VMEM((L, 8, N), jnp.float32),
            pltpu.VMEM((RPW, N), jnp.float32),
            pltpu.VMEM((RPW, N), jnp.float32),
            pltpu.VMEM((RPW, C), jnp.int32),
            pltpu.VMEM((RPW, N), jnp.float32),
            pltpu.VMEM((RPW, N), jnp.float32),
            pltpu.SemaphoreType.DMA,
        ],
    )
    sst, ld, vis_out, comp_out = f(travel_time_matrix, tw0, demand,
                                   ct, uc, vis, comp, pend, cn, a)
    return jnp.concatenate([sst[:, None], ld[:, None], vis_out, comp_out], axis=-1)


# speedup vs baseline: 45.1248x; 5.1361x over previous
import jax
import jax.numpy as jnp
from jax import lax
from jax.experimental import pallas as pl
from jax.experimental.pallas import tpu as pltpu, tpu_sc as plsc

B, N, C = 4096, 101, 50
NC, NS, L = 2, 16, 16
NW = NC * NS
RPW = B // NW
G = RPW // L


def _step_body(ttm_ref, tw_ref, dem_ref, ct_ref, uc_ref, vis_ref, comp_ref,
               pend_ref, cn_ref, a_ref,
               sst_ref, ld_ref, visout_ref, compout_ref,
               cn_v, a_v, ct_v, uc_v, sst_v, ld_v, isret_v,
               slab_v, tw_v, dem_v, pend_v, vis_v, comp_v, sem):
    wid = lax.axis_index("s") * NC + lax.axis_index("c")
    base = wid * RPW
    bsl = pl.ds(base, RPW)

    pltpu.sync_copy(cn_ref.at[bsl], cn_v)
    pltpu.sync_copy(a_ref.at[bsl], a_v)
    pltpu.sync_copy(ct_ref.at[bsl], ct_v)
    pltpu.sync_copy(uc_ref.at[bsl], uc_v)
    pltpu.sync_copy(tw_ref.at[:, bsl], tw_v)
    pltpu.sync_copy(dem_ref.at[:, bsl], dem_v)
    pltpu.sync_copy(pend_ref.at[:, bsl], pend_v)
    pltpu.sync_copy(vis_ref.at[:, bsl], vis_v)
    pltpu.sync_copy(comp_ref.at[:, bsl], comp_v)

    lanes = lax.iota(jnp.int32, L)
    zero_f = jnp.zeros((L,), jnp.float32)
    one_f = jnp.ones((L,), jnp.float32)
    zero_i = jnp.zeros((L,), jnp.int32)
    one_i = jnp.ones((L,), jnp.int32)

    for g in range(G):
        rows = g * L + lanes
        cn16 = plsc.load_gather(cn_v, [rows])
        a16 = plsc.load_gather(a_v, [rows])

        descs = []
        for j in range(L):
            cnj = cn16[j]
            a8 = pl.multiple_of((a16[j] >> 3) << 3, 8)
            descs.append(pltpu.async_copy(
                ttm_ref.at[cnj, pl.ds(a8, 8), bsl], slab_v.at[j], sem))
        for d in descs:
            d.wait()
        tt16 = plsc.load_gather(slab_v, [lanes, a16 & 7, rows])

        sw16 = plsc.load_gather(tw_v, [a16, rows])
        d16 = plsc.load_gather(dem_v, [a16, rows])
        ct16 = plsc.load_gather(ct_v, [rows])
        uc16 = plsc.load_gather(uc_v, [rows])

        is_ret = (a16 == 0) & (cn16 != 0)
        sst = jnp.maximum(ct16 + tt16, sw16)
        sst = jnp.where(is_ret, zero_f, sst)
        ld = jnp.where(is_ret, zero_f, uc16 + d16)
        plsc.store_scatter(sst_v, [rows], sst)
        plsc.store_scatter(ld_v, [rows], ld)

        nondep = jnp.where(a16 != 0, one_f, zero_f)
        plsc.store_scatter(vis_v, [a16, rows], nondep)
        isdrop = ((a16 & 1) == 0) & (a16 != 0)
        partner = jnp.maximum(a16 - 1, 0)
        plsc.store_scatter(comp_v, [a16, rows], one_f, mask=isdrop)
        plsc.store_scatter(comp_v, [partner, rows], one_f, mask=isdrop)
        plsc.store_scatter(isret_v, [rows], jnp.where(is_ret, one_i, zero_i))

    def group_body(g, carry):
        rows = g * L + lanes
        isret16 = plsc.load_gather(isret_v, [rows])
        is_ret_b = isret16 == 1

        @pl.when(jnp.max(isret16) == 1)
        def _reset():
            def col_body(c, inner):
                p16 = plsc.load_gather(pend_v, [jnp.full((L,), c, jnp.int32), rows])
                pm1 = jnp.maximum(p16 - 1, 0)
                plsc.store_scatter(vis_v, [pm1, rows], zero_f,
                                   mask=(p16 != 0) & is_ret_b)
                return inner

            lax.fori_loop(0, C, col_body, 0)

        return carry

    lax.fori_loop(0, G, group_body, 0)

    pltpu.sync_copy(sst_v, sst_ref.at[bsl])
    pltpu.sync_copy(ld_v, ld_ref.at[bsl])
    pltpu.sync_copy(vis_v, visout_ref.at[:, bsl])
    pltpu.sync_copy(comp_v, compout_ref.at[:, bsl])


def kernel(travel_time_matrix, time_windows, demand, current_time, used_capacity,
           visited, completed, pending_schedule, current_node, action):
    ttm_t = jnp.transpose(travel_time_matrix, (1, 2, 0))
    tw0_t = time_windows[:, :, 0].T
    dem_t = demand.T
    ct = current_time.reshape(-1)
    uc = used_capacity.reshape(-1)
    vis_t = visited.T.astype(jnp.float32)
    comp_t = completed.T.astype(jnp.float32)
    pend_t = pending_schedule.astype(jnp.int32).T
    cn = current_node.reshape(-1).astype(jnp.int32)
    a = action.astype(jnp.int32)

    f = pl.kernel(
        _step_body,
        out_type=[jax.ShapeDtypeStruct((B,), jnp.float32),
                  jax.ShapeDtypeStruct((B,), jnp.float32),
                  jax.ShapeDtypeStruct((N, B), jnp.float32),
                  jax.ShapeDtypeStruct((N, B), jnp.float32)],
        mesh=plsc.VectorSubcoreMesh(core_axis_name="c", subcore_axis_name="s"),
        compiler_params=pltpu.CompilerParams(use_tc_tiling_on_sc=True,
                                             needs_layout_passes=False),
        scratch_types=[
            pltpu.VMEM((RPW,), jnp.int32),
            pltpu.VMEM((RPW,), jnp.int32),
            pltpu.VMEM((RPW,), jnp.float32),
            pltpu.VMEM((RPW,), jnp.float32),
            pltpu.VMEM((RPW,), jnp.float32),
            pltpu.VMEM((RPW,), jnp.float32),
            pltpu.VMEM((RPW,), jnp.int32),
            pltpu.VMEM((L, 8, RPW), jnp.float32),
            pltpu.VMEM((N, RPW), jnp.float32),
            pltpu.VMEM((N, RPW), jnp.float32),
            pltpu.VMEM((C, RPW), jnp.int32),
            pltpu.VMEM((N, RPW), jnp.float32),
            pltpu.VMEM((N, RPW), jnp.float32),
            pltpu.SemaphoreType.DMA,
        ],
    )
    sst, ld, vis_out, comp_out = f(ttm_t, tw0_t, dem_t, ct, uc, vis_t, comp_t,
                                   pend_t, cn, a)
    out_t = jnp.concatenate([sst[None, :], ld[None, :], vis_out, comp_out], axis=0)
    return out_t.T


# speedup vs baseline: 54.0788x; 1.1984x over previous
import jax
import jax.numpy as jnp
from jax import lax
from jax.experimental import pallas as pl
from jax.experimental.pallas import tpu as pltpu, tpu_sc as plsc

B, N, C = 4096, 101, 50
NC, NS, L = 2, 16, 16
NW = NC * NS
RPW = B // NW
G = RPW // L


def _step_body(ttm_ref, tw_ref, dem_ref, ct_ref, uc_ref, vis_ref, comp_ref,
               pend_ref, cn_ref, a_ref,
               sst_ref, ld_ref, visout_ref, compout_ref,
               cn_v, a_v, ct_v, uc_v, sst_v, ld_v, isret_v,
               slab_v, tw_v, dem_v, pend_v, vis_v, comp_v,
               sem_ix, sem_st, sem_sl0, sem_sl1, sem_out):
    wid = lax.axis_index("s") * NC + lax.axis_index("c")
    base = wid * RPW
    bsl = pl.ds(base, RPW)

    dcn = pltpu.async_copy(cn_ref.at[bsl], cn_v, sem_ix)
    da = pltpu.async_copy(a_ref.at[bsl], a_v, sem_ix)
    staged = [pltpu.async_copy(ct_ref.at[bsl], ct_v, sem_st),
              pltpu.async_copy(uc_ref.at[bsl], uc_v, sem_st),
              pltpu.async_copy(tw_ref.at[:, bsl], tw_v, sem_st),
              pltpu.async_copy(dem_ref.at[:, bsl], dem_v, sem_st),
              pltpu.async_copy(pend_ref.at[:, bsl], pend_v, sem_st),
              pltpu.async_copy(vis_ref.at[:, bsl], vis_v, sem_st),
              pltpu.async_copy(comp_ref.at[:, bsl], comp_v, sem_st)]
    dcn.wait()
    da.wait()

    lanes = lax.iota(jnp.int32, L)
    zero_f = jnp.zeros((L,), jnp.float32)
    one_f = jnp.ones((L,), jnp.float32)
    zero_i = jnp.zeros((L,), jnp.int32)
    one_i = jnp.ones((L,), jnp.int32)
    slab_sems = (sem_sl0, sem_sl1)

    cn16s = [plsc.load_gather(cn_v, [g * L + lanes]) for g in range(G)]
    a16s = [plsc.load_gather(a_v, [g * L + lanes]) for g in range(G)]

    def fire_slabs(g):
        cn16, a16 = cn16s[g], a16s[g]
        descs = []
        for j in range(L):
            cnj = cn16[j]
            a8 = pl.multiple_of((a16[j] >> 3) << 3, 8)
            descs.append(pltpu.async_copy(
                ttm_ref.at[cnj, pl.ds(a8, 8), bsl], slab_v.at[g % 2, j],
                slab_sems[g % 2]))
        return descs

    inflight = fire_slabs(0)
    for g in range(G):
        nxt = fire_slabs(g + 1) if g + 1 < G else []
        for d in inflight:
            d.wait()
        inflight = nxt
        rows = g * L + lanes
        cn16, a16 = cn16s[g], a16s[g]
        tt16 = plsc.load_gather(slab_v.at[g % 2], [lanes, a16 & 7, rows])

        if g == 0:
            for d in staged:
                d.wait()
        sw16 = plsc.load_gather(tw_v, [a16, rows])
        d16 = plsc.load_gather(dem_v, [a16, rows])
        ct16 = plsc.load_gather(ct_v, [rows])
        uc16 = plsc.load_gather(uc_v, [rows])

        is_ret = (a16 == 0) & (cn16 != 0)
        sst = jnp.maximum(ct16 + tt16, sw16)
        sst = jnp.where(is_ret, zero_f, sst)
        ld = jnp.where(is_ret, zero_f, uc16 + d16)
        plsc.store_scatter(sst_v, [rows], sst)
        plsc.store_scatter(ld_v, [rows], ld)

        nondep = jnp.where(a16 != 0, one_f, zero_f)
        plsc.store_scatter(vis_v, [a16, rows], nondep)
        isdrop = ((a16 & 1) == 0) & (a16 != 0)
        partner = jnp.maximum(a16 - 1, 0)
        plsc.store_scatter(comp_v, [a16, rows], one_f, mask=isdrop)
        plsc.store_scatter(comp_v, [partner, rows], one_f, mask=isdrop)
        plsc.store_scatter(isret_v, [rows], jnp.where(is_ret, one_i, zero_i))

    outs = [pltpu.async_copy(sst_v, sst_ref.at[bsl], sem_out),
            pltpu.async_copy(ld_v, ld_ref.at[bsl], sem_out),
            pltpu.async_copy(comp_v, compout_ref.at[:, bsl], sem_out)]

    def group_body(g, carry):
        rows = g * L + lanes
        isret16 = plsc.load_gather(isret_v, [rows])
        is_ret_b = isret16 == 1

        @pl.when(jnp.max(isret16) == 1)
        def _reset():
            def col_body(c, inner):
                p16 = plsc.load_gather(pend_v, [jnp.full((L,), c, jnp.int32), rows])
                pm1 = jnp.maximum(p16 - 1, 0)
                plsc.store_scatter(vis_v, [pm1, rows], zero_f,
                                   mask=(p16 != 0) & is_ret_b)
                return inner

            lax.fori_loop(0, C, col_body, 0)

        return carry

    lax.fori_loop(0, G, group_body, 0)

    pltpu.sync_copy(vis_v, visout_ref.at[:, bsl])
    for d in outs:
        d.wait()


def kernel(travel_time_matrix, time_windows, demand, current_time, used_capacity,
           visited, completed, pending_schedule, current_node, action):
    ttm_t = jnp.transpose(travel_time_matrix, (1, 2, 0))
    tw0_t = time_windows[:, :, 0].T
    dem_t = demand.T
    ct = current_time.reshape(-1)
    uc = used_capacity.reshape(-1)
    vis_t = visited.T.astype(jnp.float32)
    comp_t = completed.T.astype(jnp.float32)
    pend_t = pending_schedule.astype(jnp.int32).T
    cn = current_node.reshape(-1).astype(jnp.int32)
    a = action.astype(jnp.int32)

    f = pl.kernel(
        _step_body,
        out_type=[jax.ShapeDtypeStruct((B,), jnp.float32),
                  jax.ShapeDtypeStruct((B,), jnp.float32),
                  jax.ShapeDtypeStruct((N, B), jnp.float32),
                  jax.ShapeDtypeStruct((N, B), jnp.float32)],
        mesh=plsc.VectorSubcoreMesh(core_axis_name="c", subcore_axis_name="s"),
        compiler_params=pltpu.CompilerParams(use_tc_tiling_on_sc=True,
                                             needs_layout_passes=False),
        scratch_types=[
            pltpu.VMEM((RPW,), jnp.int32),
            pltpu.VMEM((RPW,), jnp.int32),
            pltpu.VMEM((RPW,), jnp.float32),
            pltpu.VMEM((RPW,), jnp.float32),
            pltpu.VMEM((RPW,), jnp.float32),
            pltpu.VMEM((RPW,), jnp.float32),
            pltpu.VMEM((RPW,), jnp.int32),
            pltpu.VMEM((2, L, 8, RPW), jnp.float32),
            pltpu.VMEM((N, RPW), jnp.float32),
            pltpu.VMEM((N, RPW), jnp.float32),
            pltpu.VMEM((C, RPW), jnp.int32),
            pltpu.VMEM((N, RPW), jnp.float32),
            pltpu.VMEM((N, RPW), jnp.float32),
            pltpu.SemaphoreType.DMA,
            pltpu.SemaphoreType.DMA,
            pltpu.SemaphoreType.DMA,
            pltpu.SemaphoreType.DMA,
            pltpu.SemaphoreType.DMA,
        ],
    )
    sst, ld, vis_out, comp_out = f(ttm_t, tw0_t, dem_t, ct, uc, vis_t, comp_t,
                                   pend_t, cn, a)
    out_t = jnp.concatenate([sst[None, :], ld[None, :], vis_out, comp_out], axis=0)
    return out_t.T


# speedup vs baseline: 61.7646x; 1.1421x over previous
import jax
import jax.numpy as jnp
from jax import lax
from jax.experimental import pallas as pl
from jax.experimental.pallas import tpu as pltpu, tpu_sc as plsc

B, N, C = 4096, 101, 50
NC, NS, L = 2, 16, 16
NW = NC * NS
RPW = B // NW
G = RPW // L


def _step_body(ttm_ref, tw_ref, dem_ref, ct_ref, uc_ref, vis_ref, comp_ref,
               pend_ref, cn_ref, a_ref,
               sst_ref, ld_ref, visout_ref, compout_ref,
               cn_v, a_v, ct_v, uc_v, sst_v, ld_v, isret_v,
               slab_v, tw_v, dem_v, pend_v, vis_v, comp_v,
               sem_ix, sem_st, sem_sl0, sem_sl1, sem_out):
    wid = lax.axis_index("s") * NC + lax.axis_index("c")
    base = wid * RPW
    bsl = pl.ds(base, RPW)

    dcn = pltpu.async_copy(cn_ref.at[bsl], cn_v, sem_ix)
    da = pltpu.async_copy(a_ref.at[bsl], a_v, sem_ix)
    staged = [pltpu.async_copy(ct_ref.at[bsl], ct_v, sem_st),
              pltpu.async_copy(uc_ref.at[bsl], uc_v, sem_st),
              pltpu.async_copy(tw_ref.at[:, bsl], tw_v, sem_st),
              pltpu.async_copy(dem_ref.at[:, bsl], dem_v, sem_st),
              pltpu.async_copy(pend_ref.at[:, bsl], pend_v, sem_st),
              pltpu.async_copy(vis_ref.at[:, bsl], vis_v, sem_st),
              pltpu.async_copy(comp_ref.at[:, bsl], comp_v, sem_st)]
    dcn.wait()
    da.wait()

    lanes = lax.iota(jnp.int32, L)
    zero_f = jnp.zeros((L,), jnp.float32)
    one_f = jnp.ones((L,), jnp.float32)
    zero_i = jnp.zeros((L,), jnp.int32)
    one_i = jnp.ones((L,), jnp.int32)
    slab_sems = (sem_sl0, sem_sl1)

    cn16s = [plsc.load_gather(cn_v, [g * L + lanes]) for g in range(G)]
    a16s = [plsc.load_gather(a_v, [g * L + lanes]) for g in range(G)]

    def fire_slabs(g):
        cn16, a16 = cn16s[g], a16s[g]
        descs = []
        for j in range(L):
            descs.append(pltpu.async_copy(
                ttm_ref.at[cn16[j], a16[j], bsl], slab_v.at[g % 2, j],
                slab_sems[g % 2]))
        return descs

    inflight = fire_slabs(0)
    for g in range(G):
        nxt = fire_slabs(g + 1) if g + 1 < G else []
        for d in inflight:
            d.wait()
        inflight = nxt
        rows = g * L + lanes
        cn16, a16 = cn16s[g], a16s[g]
        tt16 = plsc.load_gather(slab_v.at[g % 2], [lanes, rows])

        if g == 0:
            for d in staged:
                d.wait()
        sw16 = plsc.load_gather(tw_v, [a16, rows])
        d16 = plsc.load_gather(dem_v, [a16, rows])
        ct16 = plsc.load_gather(ct_v, [rows])
        uc16 = plsc.load_gather(uc_v, [rows])

        is_ret = (a16 == 0) & (cn16 != 0)
        sst = jnp.maximum(ct16 + tt16, sw16)
        sst = jnp.where(is_ret, zero_f, sst)
        ld = jnp.where(is_ret, zero_f, uc16 + d16)
        plsc.store_scatter(sst_v, [rows], sst)
        plsc.store_scatter(ld_v, [rows], ld)

        nondep = jnp.where(a16 != 0, one_f, zero_f)
        plsc.store_scatter(vis_v, [a16, rows], nondep)
        isdrop = ((a16 & 1) == 0) & (a16 != 0)
        partner = jnp.maximum(a16 - 1, 0)
        plsc.store_scatter(comp_v, [a16, rows], one_f, mask=isdrop)
        plsc.store_scatter(comp_v, [partner, rows], one_f, mask=isdrop)
        plsc.store_scatter(isret_v, [rows], jnp.where(is_ret, one_i, zero_i))

    outs = [pltpu.async_copy(sst_v, sst_ref.at[bsl], sem_out),
            pltpu.async_copy(ld_v, ld_ref.at[bsl], sem_out),
            pltpu.async_copy(comp_v, compout_ref.at[:, bsl], sem_out)]

    def group_body(g, carry):
        rows = g * L + lanes
        isret16 = plsc.load_gather(isret_v, [rows])
        is_ret_b = isret16 == 1

        @pl.when(jnp.max(isret16) == 1)
        def _reset():
            def col_body(c, inner):
                p16 = plsc.load_gather(pend_v, [jnp.full((L,), c, jnp.int32), rows])
                pm1 = jnp.maximum(p16 - 1, 0)
                plsc.store_scatter(vis_v, [pm1, rows], zero_f,
                                   mask=(p16 != 0) & is_ret_b)
                return inner

            lax.fori_loop(0, C, col_body, 0)

        return carry

    lax.fori_loop(0, G, group_body, 0)

    pltpu.sync_copy(vis_v, visout_ref.at[:, bsl])
    for d in outs:
        d.wait()


def kernel(travel_time_matrix, time_windows, demand, current_time, used_capacity,
           visited, completed, pending_schedule, current_node, action):
    ttm_t = jnp.transpose(travel_time_matrix, (1, 2, 0))
    tw0_t = time_windows[:, :, 0].T
    dem_t = demand.T
    ct = current_time.reshape(-1)
    uc = used_capacity.reshape(-1)
    vis_t = visited.T.astype(jnp.float32)
    comp_t = completed.T.astype(jnp.float32)
    pend_t = pending_schedule.astype(jnp.int32).T
    cn = current_node.reshape(-1).astype(jnp.int32)
    a = action.astype(jnp.int32)

    f = pl.kernel(
        _step_body,
        out_type=[jax.ShapeDtypeStruct((B,), jnp.float32),
                  jax.ShapeDtypeStruct((B,), jnp.float32),
                  jax.ShapeDtypeStruct((N, B), jnp.float32),
                  jax.ShapeDtypeStruct((N, B), jnp.float32)],
        mesh=plsc.VectorSubcoreMesh(core_axis_name="c", subcore_axis_name="s"),
        compiler_params=pltpu.CompilerParams(use_tc_tiling_on_sc=True,
                                             needs_layout_passes=False),
        scratch_types=[
            pltpu.VMEM((RPW,), jnp.int32),
            pltpu.VMEM((RPW,), jnp.int32),
            pltpu.VMEM((RPW,), jnp.float32),
            pltpu.VMEM((RPW,), jnp.float32),
            pltpu.VMEM((RPW,), jnp.float32),
            pltpu.VMEM((RPW,), jnp.float32),
            pltpu.VMEM((RPW,), jnp.int32),
            pltpu.VMEM((2, L, RPW), jnp.float32),
            pltpu.VMEM((N, RPW), jnp.float32),
            pltpu.VMEM((N, RPW), jnp.float32),
            pltpu.VMEM((C, RPW), jnp.int32),
            pltpu.VMEM((N, RPW), jnp.float32),
            pltpu.VMEM((N, RPW), jnp.float32),
            pltpu.SemaphoreType.DMA,
            pltpu.SemaphoreType.DMA,
            pltpu.SemaphoreType.DMA,
            pltpu.SemaphoreType.DMA,
            pltpu.SemaphoreType.DMA,
        ],
    )
    sst, ld, vis_out, comp_out = f(ttm_t, tw0_t, dem_t, ct, uc, vis_t, comp_t,
                                   pend_t, cn, a)
    out_t = jnp.concatenate([sst[None, :], ld[None, :], vis_out, comp_out], axis=0)
    return out_t.T


# speedup vs baseline: 70.1375x; 1.1356x over previous
import jax
import jax.numpy as jnp
from jax import lax
from jax.experimental import pallas as pl
from jax.experimental.pallas import tpu as pltpu, tpu_sc as plsc

B, N, C = 4096, 101, 50
NC, NS, L = 2, 16, 16
NW = NC * NS
RPW = B // NW
G = RPW // L
NOUT = 2 + 2 * N


def _step_body(ttm_ref, tw_ref, dem_ref, ct_ref, uc_ref, vis_ref, comp_ref,
               pend_ref, cn_ref, a_ref,
               out_ref,
               cn_v, a_v, ct_v, uc_v, isret_v,
               slab_v, tw_v, dem_v, pend_v, out_v,
               sem_ix, sem_st, sem_sl0, sem_sl1):
    wid = lax.axis_index("s") * NC + lax.axis_index("c")
    base = wid * RPW
    bsl = pl.ds(base, RPW)

    dcn = pltpu.async_copy(cn_ref.at[bsl], cn_v, sem_ix)
    da = pltpu.async_copy(a_ref.at[bsl], a_v, sem_ix)
    staged = [pltpu.async_copy(ct_ref.at[bsl], ct_v, sem_st),
              pltpu.async_copy(uc_ref.at[bsl], uc_v, sem_st),
              pltpu.async_copy(tw_ref.at[:, :, bsl], tw_v, sem_st),
              pltpu.async_copy(dem_ref.at[:, bsl], dem_v, sem_st),
              pltpu.async_copy(pend_ref.at[:, bsl], pend_v, sem_st),
              pltpu.async_copy(vis_ref.at[:, bsl], out_v.at[pl.ds(2, N)],
                               sem_st),
              pltpu.async_copy(comp_ref.at[:, bsl], out_v.at[pl.ds(2 + N, N)],
                               sem_st)]
    dcn.wait()
    da.wait()

    lanes = lax.iota(jnp.int32, L)
    zero_f = jnp.zeros((L,), jnp.float32)
    one_f = jnp.ones((L,), jnp.float32)
    zero_i = jnp.zeros((L,), jnp.int32)
    one_i = jnp.ones((L,), jnp.int32)
    slab_sems = (sem_sl0, sem_sl1)

    cn16s = [plsc.load_gather(cn_v, [g * L + lanes]) for g in range(G)]
    a16s = [plsc.load_gather(a_v, [g * L + lanes]) for g in range(G)]

    def fire_slabs(g):
        cn16, a16 = cn16s[g], a16s[g]
        descs = []
        for j in range(L):
            descs.append(pltpu.async_copy(
                ttm_ref.at[cn16[j], a16[j], bsl], slab_v.at[g % 2, j],
                slab_sems[g % 2]))
        return descs

    inflight = fire_slabs(0)
    for g in range(G):
        nxt = fire_slabs(g + 1) if g + 1 < G else []
        for d in inflight:
            d.wait()
        inflight = nxt
        rows = g * L + lanes
        cn16, a16 = cn16s[g], a16s[g]
        tt16 = plsc.load_gather(slab_v.at[g % 2], [lanes, rows])

        if g == 0:
            for d in staged:
                d.wait()
        sw16 = plsc.load_gather(tw_v, [a16, zero_i, rows])
        d16 = plsc.load_gather(dem_v, [a16, rows])
        ct16 = plsc.load_gather(ct_v, [rows])
        uc16 = plsc.load_gather(uc_v, [rows])

        is_ret = (a16 == 0) & (cn16 != 0)
        sst = jnp.maximum(ct16 + tt16, sw16)
        sst = jnp.where(is_ret, zero_f, sst)
        ld = jnp.where(is_ret, zero_f, uc16 + d16)
        plsc.store_scatter(out_v, [zero_i, rows], sst)
        plsc.store_scatter(out_v, [one_i, rows], ld)

        nondep = jnp.where(a16 != 0, one_f, zero_f)
        plsc.store_scatter(out_v, [a16 + 2, rows], nondep)
        isdrop = ((a16 & 1) == 0) & (a16 != 0)
        partner = jnp.maximum(a16 - 1, 0)
        plsc.store_scatter(out_v, [a16 + (2 + N), rows], one_f, mask=isdrop)
        plsc.store_scatter(out_v, [partner + (2 + N), rows], one_f,
                           mask=isdrop)
        plsc.store_scatter(isret_v, [rows], jnp.where(is_ret, one_i, zero_i))

    def group_body(g, carry):
        rows = g * L + lanes
        isret16 = plsc.load_gather(isret_v, [rows])
        is_ret_b = isret16 == 1

        @pl.when(jnp.max(isret16) == 1)
        def _reset():
            def col_body(c, inner):
                p16 = plsc.load_gather(pend_v, [jnp.full((L,), c, jnp.int32), rows])
                pm1 = jnp.maximum(p16 - 1, 0)
                plsc.store_scatter(out_v, [pm1 + 2, rows], zero_f,
                                   mask=(p16 != 0) & is_ret_b)
                return inner

            lax.fori_loop(0, C, col_body, 0)

        return carry

    lax.fori_loop(0, G, group_body, 0)

    pltpu.sync_copy(out_v, out_ref.at[:, bsl])


def kernel(travel_time_matrix, time_windows, demand, current_time, used_capacity,
           visited, completed, pending_schedule, current_node, action):
    ttm_t = jnp.transpose(travel_time_matrix, (1, 2, 0))
    tw_t = jnp.transpose(time_windows, (1, 2, 0))
    dem_t = demand.T
    ct = current_time.reshape(-1)
    uc = used_capacity.reshape(-1)
    vis_t = visited.T.astype(jnp.float32)
    comp_t = completed.T.astype(jnp.float32)
    pend_t = pending_schedule.astype(jnp.int32).T
    cn = current_node.reshape(-1).astype(jnp.int32)
    a = action.astype(jnp.int32)

    f = pl.kernel(
        _step_body,
        out_type=jax.ShapeDtypeStruct((NOUT, B), jnp.float32),
        mesh=plsc.VectorSubcoreMesh(core_axis_name="c", subcore_axis_name="s"),
        compiler_params=pltpu.CompilerParams(use_tc_tiling_on_sc=True,
                                             needs_layout_passes=False),
        scratch_types=[
            pltpu.VMEM((RPW,), jnp.int32),
            pltpu.VMEM((RPW,), jnp.int32),
            pltpu.VMEM((RPW,), jnp.float32),
            pltpu.VMEM((RPW,), jnp.float32),
            pltpu.VMEM((RPW,), jnp.int32),
            pltpu.VMEM((2, L, RPW), jnp.float32),
            pltpu.VMEM((N, 2, RPW), jnp.float32),
            pltpu.VMEM((N, RPW), jnp.float32),
            pltpu.VMEM((C, RPW), jnp.int32),
            pltpu.VMEM((NOUT, RPW), jnp.float32),
            pltpu.SemaphoreType.DMA,
            pltpu.SemaphoreType.DMA,
            pltpu.SemaphoreType.DMA,
            pltpu.SemaphoreType.DMA,
        ],
    )
    out_t = f(ttm_t, tw_t, dem_t, ct, uc, vis_t, comp_t, pend_t, cn, a)
    return out_t.T


# speedup vs baseline: 73.8538x; 1.0530x over previous
import jax
import jax.numpy as jnp
from jax import lax
from jax.experimental import pallas as pl
from jax.experimental.pallas import tpu as pltpu, tpu_sc as plsc

B, N, C = 4096, 101, 50
NC, NS, L = 2, 16, 16
NW = NC * NS
RPW = B // NW
G = RPW // L
NOUT = 2 + 2 * N


def _step_body(ttm_ref, tw_ref, dem_ref, ct_ref, uc_ref, vis_ref, comp_ref,
               pend_ref, cn_ref, a_ref,
               out_ref,
               cn_v, a_v, ct_v, uc_v, isret_v,
               slab_v, tw_v, dem_v, pend_v, out_v,
               sem_ix, sem_st, sem_sl):
    wid = lax.axis_index("s") * NC + lax.axis_index("c")
    base = wid * RPW
    bsl = pl.ds(base, RPW)

    dcn = pltpu.async_copy(cn_ref.at[bsl], cn_v, sem_ix)
    da = pltpu.async_copy(a_ref.at[bsl], a_v, sem_ix)
    staged = [pltpu.async_copy(ct_ref.at[bsl], ct_v, sem_st),
              pltpu.async_copy(uc_ref.at[bsl], uc_v, sem_st),
              pltpu.async_copy(tw_ref.at[:, :, bsl], tw_v, sem_st),
              pltpu.async_copy(dem_ref.at[:, bsl], dem_v, sem_st),
              pltpu.async_copy(pend_ref.at[:, bsl], pend_v, sem_st),
              pltpu.async_copy(vis_ref.at[:, bsl], out_v.at[pl.ds(2, N)],
                               sem_st),
              pltpu.async_copy(comp_ref.at[:, bsl], out_v.at[pl.ds(2 + N, N)],
                               sem_st)]
    dcn.wait()
    da.wait()

    lanes = lax.iota(jnp.int32, L)
    zero_f = jnp.zeros((L,), jnp.float32)
    one_f = jnp.ones((L,), jnp.float32)
    zero_i = jnp.zeros((L,), jnp.int32)
    one_i = jnp.ones((L,), jnp.int32)

    cn16s = [plsc.load_gather(cn_v, [g * L + lanes]) for g in range(G)]
    a16s = [plsc.load_gather(a_v, [g * L + lanes]) for g in range(G)]

    slabs = []
    for g in range(G):
        cn16, a16 = cn16s[g], a16s[g]
        descs = []
        for j in range(L):
            descs.append(pltpu.async_copy(
                ttm_ref.at[cn16[j], a16[j], pl.ds(base + g * L, L)],
                slab_v.at[g, j], sem_sl.at[g]))
        slabs.append(descs)

    for g in range(G):
        for d in slabs[g]:
            d.wait()
        rows = g * L + lanes
        cn16, a16 = cn16s[g], a16s[g]
        tt16 = plsc.load_gather(slab_v.at[g], [lanes, lanes])

        if g == 0:
            for d in staged:
                d.wait()
        sw16 = plsc.load_gather(tw_v, [a16, zero_i, rows])
        d16 = plsc.load_gather(dem_v, [a16, rows])
        ct16 = plsc.load_gather(ct_v, [rows])
        uc16 = plsc.load_gather(uc_v, [rows])

        is_ret = (a16 == 0) & (cn16 != 0)
        sst = jnp.maximum(ct16 + tt16, sw16)
        sst = jnp.where(is_ret, zero_f, sst)
        ld = jnp.where(is_ret, zero_f, uc16 + d16)
        plsc.store_scatter(out_v, [zero_i, rows], sst)
        plsc.store_scatter(out_v, [one_i, rows], ld)

        nondep = jnp.where(a16 != 0, one_f, zero_f)
        plsc.store_scatter(out_v, [a16 + 2, rows], nondep)
        isdrop = ((a16 & 1) == 0) & (a16 != 0)
        partner = jnp.maximum(a16 - 1, 0)
        plsc.store_scatter(out_v, [a16 + (2 + N), rows], one_f, mask=isdrop)
        plsc.store_scatter(out_v, [partner + (2 + N), rows], one_f,
                           mask=isdrop)
        plsc.store_scatter(isret_v, [rows], jnp.where(is_ret, one_i, zero_i))

    def group_body(g, carry):
        rows = g * L + lanes
        isret16 = plsc.load_gather(isret_v, [rows])
        is_ret_b = isret16 == 1

        @pl.when(jnp.max(isret16) == 1)
        def _reset():
            def col_body(c, inner):
                p16 = plsc.load_gather(pend_v, [jnp.full((L,), c, jnp.int32), rows])
                pm1 = jnp.maximum(p16 - 1, 0)
                plsc.store_scatter(out_v, [pm1 + 2, rows], zero_f,
                                   mask=(p16 != 0) & is_ret_b)
                return inner

            lax.fori_loop(0, C, col_body, 0)

        return carry

    lax.fori_loop(0, G, group_body, 0)

    pltpu.sync_copy(out_v, out_ref.at[:, bsl])


def kernel(travel_time_matrix, time_windows, demand, current_time, used_capacity,
           visited, completed, pending_schedule, current_node, action):
    ttm_t = jnp.transpose(travel_time_matrix, (1, 2, 0))
    tw_t = jnp.transpose(time_windows, (1, 2, 0))
    dem_t = demand.T
    ct = current_time.reshape(-1)
    uc = used_capacity.reshape(-1)
    vis_t = visited.T.astype(jnp.float32)
    comp_t = completed.T.astype(jnp.float32)
    pend_t = pending_schedule.astype(jnp.int32).T
    cn = current_node.reshape(-1).astype(jnp.int32)
    a = action.astype(jnp.int32)

    f = pl.kernel(
        _step_body,
        out_type=jax.ShapeDtypeStruct((NOUT, B), jnp.float32),
        mesh=plsc.VectorSubcoreMesh(core_axis_name="c", subcore_axis_name="s"),
        compiler_params=pltpu.CompilerParams(use_tc_tiling_on_sc=True,
                                             needs_layout_passes=False),
        scratch_types=[
            pltpu.VMEM((RPW,), jnp.int32),
            pltpu.VMEM((RPW,), jnp.int32),
            pltpu.VMEM((RPW,), jnp.float32),
            pltpu.VMEM((RPW,), jnp.float32),
            pltpu.VMEM((RPW,), jnp.int32),
            pltpu.VMEM((G, L, L), jnp.float32),
            pltpu.VMEM((N, 2, RPW), jnp.float32),
            pltpu.VMEM((N, RPW), jnp.float32),
            pltpu.VMEM((C, RPW), jnp.int32),
            pltpu.VMEM((NOUT, RPW), jnp.float32),
            pltpu.SemaphoreType.DMA,
            pltpu.SemaphoreType.DMA,
            pltpu.SemaphoreType.DMA((G,)),
        ],
    )
    out_t = f(ttm_t, tw_t, dem_t, ct, uc, vis_t, comp_t, pend_t, cn, a)
    return out_t.T


# speedup vs baseline: 74.8886x; 1.0140x over previous
import jax
import jax.numpy as jnp
from jax import lax
from jax.experimental import pallas as pl
from jax.experimental.pallas import tpu as pltpu, tpu_sc as plsc

B, N, C = 4096, 101, 50
NC, NS, L = 2, 16, 16
NW = NC * NS
RPW = B // NW
G = RPW // L
NOUT = 2 + 2 * N


def _step_body(ttm_ref, tw_ref, dem_ref, ct_ref, uc_ref, vis_ref, comp_ref,
               pend_ref, cn_ref, a_ref,
               out_ref,
               cn_v, a_v, ct_v, uc_v, isret_v,
               slab_v, tw_v, dem_v, pend_v, out_v,
               sem_ix, sem_st, sem_sl):
    wid = lax.axis_index("s") * NC + lax.axis_index("c")
    base = wid * RPW
    bsl = pl.ds(base, RPW)

    dcn = pltpu.async_copy(cn_ref.at[bsl], cn_v, sem_ix)
    da = pltpu.async_copy(a_ref.at[bsl], a_v, sem_ix)
    staged = [pltpu.async_copy(vis_ref.at[:, bsl], out_v.at[pl.ds(2, N)],
                               sem_st),
              pltpu.async_copy(comp_ref.at[:, bsl], out_v.at[pl.ds(2 + N, N)],
                               sem_st),
              pltpu.async_copy(tw_ref.at[:, 0, bsl], tw_v, sem_st),
              pltpu.async_copy(dem_ref.at[:, bsl], dem_v, sem_st),
              pltpu.async_copy(pend_ref.at[:, bsl], pend_v, sem_st),
              pltpu.async_copy(ct_ref.at[bsl], ct_v, sem_st),
              pltpu.async_copy(uc_ref.at[bsl], uc_v, sem_st)]
    dcn.wait()
    da.wait()

    lanes = lax.iota(jnp.int32, L)
    zero_f = jnp.zeros((L,), jnp.float32)
    one_f = jnp.ones((L,), jnp.float32)
    zero_i = jnp.zeros((L,), jnp.int32)
    one_i = jnp.ones((L,), jnp.int32)

    cn16s = [plsc.load_gather(cn_v, [g * L + lanes]) for g in range(G)]
    a16s = [plsc.load_gather(a_v, [g * L + lanes]) for g in range(G)]

    slabs = []
    for g in range(G):
        cn16, a16 = cn16s[g], a16s[g]
        descs = []
        for j in range(L):
            descs.append(pltpu.async_copy(
                ttm_ref.at[cn16[j], a16[j], pl.ds(base + g * L, L)],
                slab_v.at[g, j], sem_sl.at[g]))
        slabs.append(descs)

    for g in range(G):
        for d in slabs[g]:
            d.wait()
        rows = g * L + lanes
        cn16, a16 = cn16s[g], a16s[g]
        tt16 = plsc.load_gather(slab_v.at[g], [lanes, lanes])

        if g == 0:
            for d in staged:
                d.wait()
        sw16 = plsc.load_gather(tw_v, [a16, rows])
        d16 = plsc.load_gather(dem_v, [a16, rows])
        ct16 = plsc.load_gather(ct_v, [rows])
        uc16 = plsc.load_gather(uc_v, [rows])

        is_ret = (a16 == 0) & (cn16 != 0)
        sst = jnp.maximum(ct16 + tt16, sw16)
        sst = jnp.where(is_ret, zero_f, sst)
        ld = jnp.where(is_ret, zero_f, uc16 + d16)
        plsc.store_scatter(out_v, [zero_i, rows], sst)
        plsc.store_scatter(out_v, [one_i, rows], ld)

        nondep = jnp.where(a16 != 0, one_f, zero_f)
        plsc.store_scatter(out_v, [a16 + 2, rows], nondep)
        isdrop = ((a16 & 1) == 0) & (a16 != 0)
        partner = jnp.maximum(a16 - 1, 0)
        plsc.store_scatter(out_v, [a16 + (2 + N), rows], one_f, mask=isdrop)
        plsc.store_scatter(out_v, [partner + (2 + N), rows], one_f,
                           mask=isdrop)
        plsc.store_scatter(isret_v, [rows], jnp.where(is_ret, one_i, zero_i))

    def group_body(g, carry):
        rows = g * L + lanes
        isret16 = plsc.load_gather(isret_v, [rows])
        is_ret_b = isret16 == 1

        @pl.when(jnp.max(isret16) == 1)
        def _reset():
            def col_body(c, inner):
                p16 = plsc.load_gather(pend_v, [jnp.full((L,), c, jnp.int32), rows])
                pm1 = jnp.maximum(p16 - 1, 0)
                plsc.store_scatter(out_v, [pm1 + 2, rows], zero_f,
                                   mask=(p16 != 0) & is_ret_b)
                return inner

            lax.fori_loop(0, C, col_body, 0)

        return carry

    lax.fori_loop(0, G, group_body, 0)

    pltpu.sync_copy(out_v, out_ref.at[:, bsl])


def kernel(travel_time_matrix, time_windows, demand, current_time, used_capacity,
           visited, completed, pending_schedule, current_node, action):
    ttm_t = jnp.transpose(travel_time_matrix, (1, 2, 0))
    tw_t = jnp.transpose(time_windows, (1, 2, 0))
    dem_t = demand.T
    ct = current_time.reshape(-1)
    uc = used_capacity.reshape(-1)
    vis_t = visited.T.astype(jnp.float32)
    comp_t = completed.T.astype(jnp.float32)
    pend_t = pending_schedule.astype(jnp.int32).T
    cn = current_node.reshape(-1).astype(jnp.int32)
    a = action.astype(jnp.int32)

    f = pl.kernel(
        _step_body,
        out_type=jax.ShapeDtypeStruct((NOUT, B), jnp.float32),
        mesh=plsc.VectorSubcoreMesh(core_axis_name="c", subcore_axis_name="s"),
        compiler_params=pltpu.CompilerParams(use_tc_tiling_on_sc=True,
                                             needs_layout_passes=False),
        scratch_types=[
            pltpu.VMEM((RPW,), jnp.int32),
            pltpu.VMEM((RPW,), jnp.int32),
            pltpu.VMEM((RPW,), jnp.float32),
            pltpu.VMEM((RPW,), jnp.float32),
            pltpu.VMEM((RPW,), jnp.int32),
            pltpu.VMEM((G, L, L), jnp.float32),
            pltpu.VMEM((N, RPW), jnp.float32),
            pltpu.VMEM((N, RPW), jnp.float32),
            pltpu.VMEM((C, RPW), jnp.int32),
            pltpu.VMEM((NOUT, RPW), jnp.float32),
            pltpu.SemaphoreType.DMA,
            pltpu.SemaphoreType.DMA,
            pltpu.SemaphoreType.DMA((G,)),
        ],
    )
    out_t = f(ttm_t, tw_t, dem_t, ct, uc, vis_t, comp_t, pend_t, cn, a)
    return out_t.T


# speedup vs baseline: 78.7487x; 1.0515x over previous
import jax
import jax.numpy as jnp
from jax import lax
from jax.experimental import pallas as pl
from jax.experimental.pallas import tpu as pltpu, tpu_sc as plsc

B, N, C = 4096, 101, 50
NC, NS, L = 2, 16, 16
NW = NC * NS
RPW = B // NW
G = RPW // L
NOUT = 2 + 2 * N


def _step_body(ttm_ref, tw_ref, dem_ref, ct_ref, uc_ref, vis_ref, comp_ref,
               pend_ref, cn_ref, a_ref,
               out_ref,
               cn_v, a_v, ct_v, uc_v, ret_v,
               slab_v, tw_v, dem_v, pend_v, out_v,
               sem_ix, sem_st, sem_sl):
    wid = lax.axis_index("s") * NC + lax.axis_index("c")
    base = wid * RPW
    bsl = pl.ds(base, RPW)

    dcn = pltpu.async_copy(cn_ref.at[bsl], cn_v, sem_ix)
    da = pltpu.async_copy(a_ref.at[bsl], a_v, sem_ix)
    staged = [pltpu.async_copy(vis_ref.at[:, bsl], out_v.at[pl.ds(2, N)],
                               sem_st),
              pltpu.async_copy(comp_ref.at[:, bsl], out_v.at[pl.ds(2 + N, N)],
                               sem_st),
              pltpu.async_copy(tw_ref.at[:, 0, bsl], tw_v, sem_st),
              pltpu.async_copy(dem_ref.at[:, bsl], dem_v, sem_st),
              pltpu.async_copy(pend_ref.at[:, bsl], pend_v, sem_st),
              pltpu.async_copy(ct_ref.at[bsl], ct_v, sem_st),
              pltpu.async_copy(uc_ref.at[bsl], uc_v, sem_st)]
    dcn.wait()
    da.wait()

    lanes = lax.iota(jnp.int32, L)
    zero_f = jnp.zeros((L,), jnp.float32)
    one_f = jnp.ones((L,), jnp.float32)
    zero_i = jnp.zeros((L,), jnp.int32)
    one_i = jnp.ones((L,), jnp.int32)

    cn16s = [plsc.load_gather(cn_v, [g * L + lanes]) for g in range(G)]
    a16s = [plsc.load_gather(a_v, [g * L + lanes]) for g in range(G)]
    cnt = jnp.int32(0)

    slabs = []
    for g in range(G):
        cn16, a16 = cn16s[g], a16s[g]
        descs = []
        for j in range(L):
            descs.append(pltpu.async_copy(
                ttm_ref.at[cn16[j], a16[j], pl.ds(base + g * L, L)],
                slab_v.at[g, j], sem_sl.at[g]))
        slabs.append(descs)

    for g in range(G):
        for d in slabs[g]:
            d.wait()
        rows = g * L + lanes
        cn16, a16 = cn16s[g], a16s[g]
        tt16 = plsc.load_gather(slab_v.at[g], [lanes, lanes])

        if g == 0:
            for d in staged:
                d.wait()
        sw16 = plsc.load_gather(tw_v, [a16, rows])
        d16 = plsc.load_gather(dem_v, [a16, rows])
        ct16 = plsc.load_gather(ct_v, [rows])
        uc16 = plsc.load_gather(uc_v, [rows])

        is_ret = (a16 == 0) & (cn16 != 0)
        sst = jnp.maximum(ct16 + tt16, sw16)
        sst = jnp.where(is_ret, zero_f, sst)
        ld = jnp.where(is_ret, zero_f, uc16 + d16)
        plsc.store_scatter(out_v, [zero_i, rows], sst)
        plsc.store_scatter(out_v, [one_i, rows], ld)

        nondep = jnp.where(a16 != 0, one_f, zero_f)
        plsc.store_scatter(out_v, [a16 + 2, rows], nondep)
        isdrop = ((a16 & 1) == 0) & (a16 != 0)
        partner = jnp.maximum(a16 - 1, 0)
        plsc.store_scatter(out_v, [a16 + (2 + N), rows], one_f, mask=isdrop)
        plsc.store_scatter(out_v, [partner + (2 + N), rows], one_f,
                           mask=isdrop)
        plsc.store_compressed(ret_v.at[pl.ds(cnt, L)], rows, mask=is_ret)
        cnt = cnt + jnp.sum(jnp.where(is_ret, one_i, zero_i))

    def row_body(i, carry):
        b16 = plsc.load_gather(ret_v, [jnp.full((L,), i, jnp.int32)])
        for k in range((C + L - 1) // L):
            cidx = lanes + k * L
            if (k + 1) * L <= C:
                p16 = plsc.load_gather(pend_v, [cidx, b16])
                m = p16 != 0
            else:
                cm = cidx < C
                p16 = plsc.load_gather(pend_v, [jnp.where(cm, cidx, zero_i),
                                                b16])
                m = cm & (p16 != 0)
            pm1 = jnp.maximum(p16 - 1, 0)
            plsc.store_scatter(out_v, [pm1 + 2, b16], zero_f, mask=m)
        return carry

    lax.fori_loop(0, cnt, row_body, 0)

    pltpu.sync_copy(out_v, out_ref.at[:, bsl])


def kernel(travel_time_matrix, time_windows, demand, current_time, used_capacity,
           visited, completed, pending_schedule, current_node, action):
    ttm_t = jnp.transpose(travel_time_matrix, (1, 2, 0))
    tw_t = jnp.transpose(time_windows, (1, 2, 0))
    dem_t = demand.T
    ct = current_time.reshape(-1)
    uc = used_capacity.reshape(-1)
    vis_t = visited.T.astype(jnp.float32)
    comp_t = completed.T.astype(jnp.float32)
    pend_t = pending_schedule.astype(jnp.int32).T
    cn = current_node.reshape(-1).astype(jnp.int32)
    a = action.astype(jnp.int32)

    f = pl.kernel(
        _step_body,
        out_type=jax.ShapeDtypeStruct((NOUT, B), jnp.float32),
        mesh=plsc.VectorSubcoreMesh(core_axis_name="c", subcore_axis_name="s"),
        compiler_params=pltpu.CompilerParams(use_tc_tiling_on_sc=True,
                                             needs_layout_passes=False),
        scratch_types=[
            pltpu.VMEM((RPW,), jnp.int32),
            pltpu.VMEM((RPW,), jnp.int32),
            pltpu.VMEM((RPW,), jnp.float32),
            pltpu.VMEM((RPW,), jnp.float32),
            pltpu.VMEM((RPW + L,), jnp.int32),
            pltpu.VMEM((G, L, L), jnp.float32),
            pltpu.VMEM((N, RPW), jnp.float32),
            pltpu.VMEM((N, RPW), jnp.float32),
            pltpu.VMEM((C, RPW), jnp.int32),
            pltpu.VMEM((NOUT, RPW), jnp.float32),
            pltpu.SemaphoreType.DMA,
            pltpu.SemaphoreType.DMA,
            pltpu.SemaphoreType.DMA((G,)),
        ],
    )
    out_t = f(ttm_t, tw_t, dem_t, ct, uc, vis_t, comp_t, pend_t, cn, a)
    return out_t.T
